# Initial kernel scaffold; baseline (speedup 1.0000x reference)
import jax
import jax.numpy as jnp
from jax.experimental import pallas as pl


def kernel(input, gdt_ts):
    raise NotImplementedError("write your pallas kernel here")


# speedup vs baseline: 1.3725x; 1.3725x over previous
import jax
import jax.numpy as jnp
from jax.experimental import pallas as pl
from jax.experimental.pallas import tpu as pltpu

_GAP = 1.0
_THR = 0.1
_D = 256
_GB = 8
_CORES = 2


def _loss_kernel(o_ref, t_ref, out_ref, acc_ref, *, nsteps, g_valid, scale):
    j = pl.program_id(1)
    core = pl.program_id(0)
    blk = core * nsteps + j

    o = o_ref[...]
    t = t_ref[...]

    row = blk * _GB + jax.lax.broadcasted_iota(jnp.int32, (_GB, 1), 0)
    t = jnp.where(row < g_valid, t, 0.0)

    oi = (o + (_GAP * 0.5))[:, :, None]
    oj = (o - (_GAP * 0.5))[:, None, :]
    ti = t[:, :, None]
    tj = (t + _THR)[:, None, :]

    m = ti > tj
    z = jnp.maximum(oi - oj, 0.0)
    s = jnp.where(m, z, 0.0)

    part = jnp.sum(s, axis=(0, 1))

    @pl.when(j == 0)
    def _init():
        acc_ref[...] = jnp.zeros_like(acc_ref)

    acc_ref[...] += part.reshape(1, _D)

    @pl.when(j == nsteps - 1)
    def _fin():
        out_ref[...] = (jnp.sum(acc_ref[...]) * scale).reshape(1, 1, 1)


def kernel(input, gdt_ts):
    B = input.shape[0]
    K = B // _D
    G = K - 1
    N = G * _D * (_D - 1)

    o2 = input.reshape(K, _D)
    t2 = gdt_ts.reshape(K, _D)

    nsteps = K // (_CORES * _GB)
    scale = 2.0 / float(N)

    import functools
    body = functools.partial(_loss_kernel, nsteps=nsteps, g_valid=G,
                             scale=scale)

    parts = pl.pallas_call(
        body,
        grid=(_CORES, nsteps),
        in_specs=[
            pl.BlockSpec((_GB, _D), lambda i, j: (i * (K // (_CORES * _GB)) + j, 0)),
            pl.BlockSpec((_GB, _D), lambda i, j: (i * (K // (_CORES * _GB)) + j, 0)),
        ],
        out_specs=pl.BlockSpec((1, 1, 1), lambda i, j: (i, 0, 0)),
        out_shape=jax.ShapeDtypeStruct((_CORES, 1, 1), jnp.float32),
        scratch_shapes=[pltpu.VMEM((1, _D), jnp.float32)],
        compiler_params=pltpu.CompilerParams(
            dimension_semantics=("parallel", "arbitrary"),
        ),
    )(o2, t2)

    return jnp.sum(parts).reshape(1)
